# Initial kernel scaffold; baseline (speedup 1.0000x reference)
import functools

import jax
import jax.numpy as jnp
from jax import lax
from jax.experimental import pallas as pl
from jax.experimental.pallas import tpu as pltpu
from jax.experimental.pallas import tpu_sc as plsc

N = 10000
D = 101
E = 640000

NPAD = 10240
DPAD = 112
NROWS16 = NPAD // 16

NE = E + N
EPAD = 655360
ROWS_E = EPAD // 128
PAD_NODE = 10016

NC = 2
NS = 16
NW = NC * NS
ET_ROWS = ROWS_E // NW
CA_ROWS = 16
CB_ROWS = 4
R_BLK = 1280

_F32 = jnp.float32
_I32 = jnp.int32


def _mat_body_single(in_ref, w_ref, a8_ref, h_ref, asd_ref):
  s = in_ref[...]
  h = jnp.dot(s, w_ref[...], preferred_element_type=_F32,
              precision=lax.Precision.HIGHEST)
  h_ref[...] = h
  asd_ref[...] = lax.dot_general(
      a8_ref[...], h, (((0,), (1,)), ((), ())),
      preferred_element_type=_F32, precision=lax.Precision.HIGHEST)


def _mat_body_pair(acc_ref, b_ref, w_ref, a8_ref, h_ref, asd_ref):
  s = acc_ref[0] + acc_ref[1] + b_ref[...]
  h = jnp.dot(s, w_ref[...], preferred_element_type=_F32,
              precision=lax.Precision.HIGHEST)
  h_ref[...] = h
  asd_ref[...] = lax.dot_general(
      a8_ref[...], h, (((0,), (1,)), ((), ())),
      preferred_element_type=_F32, precision=lax.Precision.HIGHEST)


def _mat_single(x_pad, w_pad, a8):
  grid = NPAD // R_BLK
  return pl.pallas_call(
      _mat_body_single,
      grid=(grid,),
      in_specs=[
          pl.BlockSpec((R_BLK, DPAD), lambda i: (i, 0)),
          pl.BlockSpec((DPAD, DPAD), lambda i: (0, 0)),
          pl.BlockSpec((DPAD, 8), lambda i: (0, 0)),
      ],
      out_specs=[
          pl.BlockSpec((R_BLK, DPAD), lambda i: (i, 0)),
          pl.BlockSpec((8, R_BLK), lambda i: (0, i)),
      ],
      out_shape=[
          jax.ShapeDtypeStruct((NPAD, DPAD), _F32),
          jax.ShapeDtypeStruct((8, NPAD), _F32),
      ],
  )(x_pad, w_pad, a8)


def _mat_pair(acc, b_pad, w_pad, a8):
  grid = NPAD // R_BLK
  return pl.pallas_call(
      _mat_body_pair,
      grid=(grid,),
      in_specs=[
          pl.BlockSpec((2, R_BLK, DPAD), lambda i: (0, i, 0)),
          pl.BlockSpec((1, DPAD), lambda i: (0, 0)),
          pl.BlockSpec((DPAD, DPAD), lambda i: (0, 0)),
          pl.BlockSpec((DPAD, 8), lambda i: (0, 0)),
      ],
      out_specs=[
          pl.BlockSpec((R_BLK, DPAD), lambda i: (i, 0)),
          pl.BlockSpec((8, R_BLK), lambda i: (0, i)),
      ],
      out_shape=[
          jax.ShapeDtypeStruct((NPAD, DPAD), _F32),
          jax.ShapeDtypeStruct((8, NPAD), _F32),
      ],
  )(acc, b_pad, w_pad, a8)


def _final_body(acc_ref, b_ref, out_ref):
  out_ref[...] = acc_ref[0] + acc_ref[1] + b_ref[...]


def _final_combine(acc, b_pad):
  grid = NPAD // R_BLK
  return pl.pallas_call(
      _final_body,
      grid=(grid,),
      in_specs=[
          pl.BlockSpec((2, R_BLK, DPAD), lambda i: (0, i, 0)),
          pl.BlockSpec((1, DPAD), lambda i: (0, 0)),
      ],
      out_specs=pl.BlockSpec((R_BLK, DPAD), lambda i: (i, 0)),
      out_shape=jax.ShapeDtypeStruct((NPAD, DPAD), _F32),
  )(acc, b_pad)


_SC_MESH = plsc.VectorSubcoreMesh(core_axis_name="c", subcore_axis_name="s")


@functools.partial(
    pl.kernel,
    out_type=(
        jax.ShapeDtypeStruct((ROWS_E, 128), _F32),
        jax.ShapeDtypeStruct((NC, NROWS16, 16), _F32),
    ),
    mesh=_SC_MESH,
    scratch_types=[
        pltpu.VMEM((NPAD,), _F32),
        pltpu.VMEM((NPAD,), _F32),
        pltpu.VMEM((NROWS16, 16), _F32),
        pltpu.VMEM((5, 128), _I32),
        pltpu.VMEM((CA_ROWS, 128), _I32),
        pltpu.VMEM((CA_ROWS, 128), _I32),
        pltpu.VMEM((CA_ROWS, 128), _F32),
        pltpu.VMEM_SHARED((NROWS16, 16), _F32),
    ],
)
def _sc_edge_scalar(asd_hbm, src_hbm, dst_hbm, ex_hbm, den_hbm,
                    as_v, ad_v, den_v, ramp_v, src_c, dst_c, ex_c, den_sh):
  cidx = lax.axis_index("c")
  sidx = lax.axis_index("s")
  wid = sidx * NC + cidx

  pltpu.sync_copy(asd_hbm.at[0], as_v)
  pltpu.sync_copy(asd_hbm.at[1], ad_v)

  zero16 = jnp.zeros((16,), _F32)

  def _zero_row(r, _):
    den_v[r] = zero16
    return 0
  lax.fori_loop(0, NROWS16, _zero_row, 0)

  iota16 = lax.iota(_I32, 16)
  for j in range(5):
    for q in range(8):
      ramp_v[j, pl.ds(q * 16, 16)] = iota16 + (j * 128 + q * 16)

  base_row = wid * ET_ROWS
  for ci in range(ET_ROWS // CA_ROWS):
    rb = base_row + ci * CA_ROWS
    pltpu.sync_copy(src_hbm.at[pl.ds(rb, CA_ROWS)], src_c)
    pltpu.sync_copy(dst_hbm.at[pl.ds(rb, CA_ROWS)], dst_c)

    def _row(r, _):
      for q in range(8):
        s16 = src_c[r, pl.ds(q * 16, 16)]
        d16 = dst_c[r, pl.ds(q * 16, 16)]
        av = plsc.load_gather(as_v, [s16])
        bv = plsc.load_gather(ad_v, [d16])
        e = av + bv
        e = jnp.where(e > 0.0, e, 0.2 * e)
        exv = jnp.exp(e)
        ex_c[r, pl.ds(q * 16, 16)] = exv
        plsc.addupdate_scatter(
            den_v,
            [lax.shift_right_logical(d16, 4), lax.bitwise_and(d16, 15)],
            exv)
      return 0
    lax.fori_loop(0, CA_ROWS, _row, 0)

    pltpu.sync_copy(ex_c, ex_hbm.at[pl.ds(rb, CA_ROWS)])

  @pl.when(sidx == 0)
  def _():
    pltpu.sync_copy(den_v, den_sh)
  plsc.subcore_barrier()

  @pl.when(sidx != 0)
  def _():
    for j in range(5):
      pltpu.sync_copy(den_v.at[pl.ds(j * 128, 128)],
                      den_sh.at[ramp_v.at[j]], add=True)
  plsc.subcore_barrier()

  @pl.when(sidx == 0)
  def _():
    pltpu.sync_copy(den_sh, den_hbm.at[cidx])


@functools.partial(
    pl.kernel,
    out_type=jax.ShapeDtypeStruct((NC, NPAD, DPAD), _F32),
    mesh=_SC_MESH,
    scratch_types=[
        pltpu.VMEM((NROWS16, 16), _F32),
        pltpu.VMEM((NROWS16, 16), _F32),
        pltpu.VMEM((CB_ROWS, 128), _I32),
        pltpu.VMEM((CB_ROWS, 128), _I32),
        pltpu.VMEM((CB_ROWS, 128), _F32),
        pltpu.VMEM((CB_ROWS * 128,), _F32),
        pltpu.VMEM((CB_ROWS * 128, DPAD), _F32),
        pltpu.VMEM((32, DPAD), _F32),
        pltpu.VMEM_SHARED((NPAD, DPAD), _F32),
        pltpu.SemaphoreType.DMA,
    ],
)
def _sc_edge_rows(h_hbm, src_hbm, dst_hbm, ex_hbm, den_hbm, out_hbm,
                  den_v, den2_v, src_c, dst_c, ex_c, w_v, rows_v, zer_v,
                  acc_sh, sem):
  cidx = lax.axis_index("c")
  sidx = lax.axis_index("s")
  wid = sidx * NC + cidx

  pltpu.sync_copy(den_hbm.at[0], den_v)
  pltpu.sync_copy(den_hbm.at[1], den2_v)

  def _comb(r, _):
    den_v[r] = den_v[r] + den2_v[r]
    return 0
  lax.fori_loop(0, NROWS16, _comb, 0)

  zero16 = jnp.zeros((16,), _F32)
  for r in range(32):
    for q in range(DPAD // 16):
      zer_v[r, pl.ds(q * 16, 16)] = zero16

  stripe = sidx * (NPAD // NS)
  for t in range((NPAD // NS) // 32):
    pltpu.sync_copy(zer_v, acc_sh.at[pl.ds(stripe + t * 32, 32)])
  plsc.subcore_barrier()

  base_row = wid * ET_ROWS

  def _chunk(ci, _):
    rb = base_row + ci * CB_ROWS
    pltpu.sync_copy(src_hbm.at[pl.ds(rb, CB_ROWS)], src_c)
    pltpu.sync_copy(dst_hbm.at[pl.ds(rb, CB_ROWS)], dst_c)
    pltpu.sync_copy(ex_hbm.at[pl.ds(rb, CB_ROWS)], ex_c)

    descs = []
    for j in range(CB_ROWS):
      descs.append(pltpu.async_copy(
          h_hbm.at[src_c.at[j]], rows_v.at[pl.ds(j * 128, 128)], sem))
    for d in descs:
      d.wait()

    for r in range(CB_ROWS):
      for q in range(8):
        d16 = dst_c[r, pl.ds(q * 16, 16)]
        dv = plsc.load_gather(
            den_v,
            [lax.shift_right_logical(d16, 4), lax.bitwise_and(d16, 15)])
        exv = ex_c[r, pl.ds(q * 16, 16)]
        w_v[pl.ds((r * 8 + q) * 16, 16)] = exv / dv

    def _scale(e, _):
      wb = plsc.load_gather(w_v, [jnp.full((16,), e, _I32)])
      for q in range(DPAD // 16):
        rows_v[e, pl.ds(q * 16, 16)] = rows_v[e, pl.ds(q * 16, 16)] * wb
      return 0
    lax.fori_loop(0, CB_ROWS * 128, _scale, 0)

    for j in range(CB_ROWS):
      pltpu.sync_copy(rows_v.at[pl.ds(j * 128, 128)],
                      acc_sh.at[dst_c.at[j]], add=True)
    return 0

  lax.fori_loop(0, ET_ROWS // CB_ROWS, _chunk, 0)

  plsc.subcore_barrier()
  pltpu.sync_copy(acc_sh.at[pl.ds(stripe, NPAD // NS)],
                  out_hbm.at[cidx, pl.ds(stripe, NPAD // NS)])


def kernel(x, edge_index, W1, a_s1, a_d1, b1, W2, a_s2, a_d2, b2):
  x_pad = jnp.zeros((NPAD, DPAD), _F32).at[:N, :D].set(x)

  loop = jnp.arange(N, dtype=_I32)
  padv = jnp.full((EPAD - NE,), PAD_NODE, _I32)
  src = jnp.concatenate([edge_index[0], loop, padv]).reshape(ROWS_E, 128)
  dst = jnp.concatenate([edge_index[1], loop, padv]).reshape(ROWS_E, 128)

  def pad_w(w):
    return jnp.zeros((DPAD, DPAD), _F32).at[:D, :D].set(w)

  def pad_a8(a_s, a_d):
    return (jnp.zeros((DPAD, 8), _F32)
            .at[:D, 0].set(a_s).at[:D, 1].set(a_d))

  def pad_b(b):
    return jnp.zeros((1, DPAD), _F32).at[0, :D].set(b)

  h1, asd1 = _mat_single(x_pad, pad_w(W1), pad_a8(a_s1, a_d1))
  ex1, den1 = _sc_edge_scalar(asd1, src, dst)
  acc1 = _sc_edge_rows(h1, src, dst, ex1, den1)

  h2, asd2 = _mat_pair(acc1, pad_b(b1), pad_w(W2), pad_a8(a_s2, a_d2))
  ex2, den2 = _sc_edge_scalar(asd2, src, dst)
  acc2 = _sc_edge_rows(h2, src, dst, ex2, den2)

  out = _final_combine(acc2, pad_b(b2))
  return out[:N, :D]


# speedup vs baseline: 15.8192x; 15.8192x over previous
import functools

import jax
import jax.numpy as jnp
from jax import lax
from jax.experimental import pallas as pl
from jax.experimental.pallas import tpu as pltpu
from jax.experimental.pallas import tpu_sc as plsc

N = 10000
D = 101
E = 640000

NPAD = 10240
DPAD = 128
DEN_R = NPAD // 128
DEN_RP = 128

NE = E + N
EPAD = 655360
ROWS_E = EPAD // 128
PAD_NODE = 10016

NC = 2
NS = 16
NW = NC * NS
ET_ROWS = ROWS_E // NW
CA_ROWS = 16
CB_ROWS = 2
R_BLK = 1280

_F32 = jnp.float32
_I32 = jnp.int32


def _mat_body_single(in_ref, w_ref, a8_ref, h_ref, asd_ref):
  s = in_ref[...]
  h = jnp.dot(s, w_ref[...], preferred_element_type=_F32,
              precision=lax.Precision.HIGHEST)
  h_ref[...] = h
  asd_ref[...] = lax.dot_general(
      a8_ref[...], h, (((0,), (1,)), ((), ())),
      preferred_element_type=_F32, precision=lax.Precision.HIGHEST)


def _mat_body_pair(acc_ref, b_ref, w_ref, a8_ref, h_ref, asd_ref):
  s = acc_ref[...] + b_ref[...]
  h = jnp.dot(s, w_ref[...], preferred_element_type=_F32,
              precision=lax.Precision.HIGHEST)
  h_ref[...] = h
  asd_ref[...] = lax.dot_general(
      a8_ref[...], h, (((0,), (1,)), ((), ())),
      preferred_element_type=_F32, precision=lax.Precision.HIGHEST)


def _mat_single(x_pad, w_pad, a8):
  grid = NPAD // R_BLK
  return pl.pallas_call(
      _mat_body_single,
      grid=(grid,),
      in_specs=[
          pl.BlockSpec((R_BLK, DPAD), lambda i: (i, 0)),
          pl.BlockSpec((DPAD, DPAD), lambda i: (0, 0)),
          pl.BlockSpec((DPAD, 8), lambda i: (0, 0)),
      ],
      out_specs=[
          pl.BlockSpec((R_BLK, DPAD), lambda i: (i, 0)),
          pl.BlockSpec((8, R_BLK), lambda i: (0, i)),
      ],
      out_shape=[
          jax.ShapeDtypeStruct((NPAD, DPAD), _F32),
          jax.ShapeDtypeStruct((8, NPAD), _F32),
      ],
  )(x_pad, w_pad, a8)


def _mat_pair(acc, b_pad, w_pad, a8):
  grid = NPAD // R_BLK
  return pl.pallas_call(
      _mat_body_pair,
      grid=(grid,),
      in_specs=[
          pl.BlockSpec((R_BLK, DPAD), lambda i: (i, 0)),
          pl.BlockSpec((1, DPAD), lambda i: (0, 0)),
          pl.BlockSpec((DPAD, DPAD), lambda i: (0, 0)),
          pl.BlockSpec((DPAD, 8), lambda i: (0, 0)),
      ],
      out_specs=[
          pl.BlockSpec((R_BLK, DPAD), lambda i: (i, 0)),
          pl.BlockSpec((8, R_BLK), lambda i: (0, i)),
      ],
      out_shape=[
          jax.ShapeDtypeStruct((NPAD, DPAD), _F32),
          jax.ShapeDtypeStruct((8, NPAD), _F32),
      ],
  )(acc, b_pad, w_pad, a8)


def _final_body(acc_ref, b_ref, out_ref):
  out_ref[...] = acc_ref[...] + b_ref[...]


def _final_combine(acc, b_pad):
  grid = NPAD // R_BLK
  return pl.pallas_call(
      _final_body,
      grid=(grid,),
      in_specs=[
          pl.BlockSpec((R_BLK, DPAD), lambda i: (i, 0)),
          pl.BlockSpec((1, DPAD), lambda i: (0, 0)),
      ],
      out_specs=pl.BlockSpec((R_BLK, DPAD), lambda i: (i, 0)),
      out_shape=jax.ShapeDtypeStruct((NPAD, DPAD), _F32),
  )(acc, b_pad)


_SC_MESH = plsc.VectorSubcoreMesh(core_axis_name="c", subcore_axis_name="s")


@functools.partial(
    pl.kernel,
    out_type=(
        jax.ShapeDtypeStruct((ROWS_E, 128), _F32),
        jax.ShapeDtypeStruct((NC * DEN_RP, 128), _F32),
    ),
    mesh=_SC_MESH,
    compiler_params=pltpu.CompilerParams(needs_layout_passes=False),
    scratch_types=[
        pltpu.VMEM((NPAD,), _F32),
        pltpu.VMEM((NPAD,), _F32),
        pltpu.VMEM((DEN_RP, 128), _F32),
        pltpu.VMEM((1, 128), _I32),
        pltpu.VMEM((CA_ROWS, 128), _I32),
        pltpu.VMEM((CA_ROWS, 128), _I32),
        pltpu.VMEM((CA_ROWS, 128), _F32),
        pltpu.VMEM_SHARED((DEN_RP, 128), _F32),
    ],
)
def _sc_edge_scalar(asd_hbm, src_hbm, dst_hbm, ex_hbm, den_hbm,
                    as_v, ad_v, den_v, ramp_v, src_c, dst_c, ex_c, den_sh):
  cidx = lax.axis_index("c")
  sidx = lax.axis_index("s")
  wid = sidx * NC + cidx

  pltpu.sync_copy(asd_hbm.at[0], as_v)
  pltpu.sync_copy(asd_hbm.at[1], ad_v)

  zero16 = jnp.zeros((16,), _F32)

  def _zero_row(r, _):
    for q in range(8):
      den_v[r, pl.ds(q * 16, 16)] = zero16
    return 0
  lax.fori_loop(0, DEN_RP, _zero_row, 0)

  iota16 = lax.iota(_I32, 16)
  for q in range(8):
    ramp_v[0, pl.ds(q * 16, 16)] = iota16 + q * 16

  base_row = wid * ET_ROWS
  for ci in range(ET_ROWS // CA_ROWS):
    rb = base_row + ci * CA_ROWS
    pltpu.sync_copy(src_hbm.at[pl.ds(rb, CA_ROWS)], src_c)
    pltpu.sync_copy(dst_hbm.at[pl.ds(rb, CA_ROWS)], dst_c)

    def _row(r, _):
      for q in range(8):
        s16 = src_c[r, pl.ds(q * 16, 16)]
        d16 = dst_c[r, pl.ds(q * 16, 16)]
        av = plsc.load_gather(as_v, [s16])
        bv = plsc.load_gather(ad_v, [d16])
        e = av + bv
        e = jnp.where(e > 0.0, e, 0.2 * e)
        exv = jnp.exp(e)
        ex_c[r, pl.ds(q * 16, 16)] = exv
        plsc.addupdate_scatter(
            den_v,
            [lax.shift_right_logical(d16, 7), lax.bitwise_and(d16, 127)],
            exv)
      return 0
    lax.fori_loop(0, CA_ROWS, _row, 0)

    pltpu.sync_copy(ex_c, ex_hbm.at[pl.ds(rb, CA_ROWS)])

  @pl.when(sidx == 0)
  def _():
    pltpu.sync_copy(den_v, den_sh)
  plsc.subcore_barrier()

  @pl.when(sidx != 0)
  def _():
    pltpu.sync_copy(den_v, den_sh.at[ramp_v.at[0]], add=True)
  plsc.subcore_barrier()

  @pl.when(sidx == 0)
  def _():
    pltpu.sync_copy(den_sh, den_hbm.at[pl.ds(cidx * DEN_RP, DEN_RP)])


_SC_MESH1 = plsc.VectorSubcoreMesh(
    core_axis_name="c", subcore_axis_name="s", num_cores=1)


@functools.partial(
    pl.kernel,
    out_type=jax.ShapeDtypeStruct((NPAD, DPAD), _F32),
    mesh=_SC_MESH1,
    compiler_params=pltpu.CompilerParams(needs_layout_passes=False),
    scratch_types=[
        pltpu.VMEM((DEN_R, 128), _F32),
        pltpu.VMEM((CB_ROWS, 128), _I32),
        pltpu.VMEM((CB_ROWS, 128), _I32),
        pltpu.VMEM((CB_ROWS, 128), _F32),
        pltpu.VMEM((CB_ROWS * 128,), _F32),
        pltpu.VMEM((CB_ROWS * 128, DPAD), _F32),
        pltpu.VMEM((16, DPAD), _F32),
        pltpu.VMEM_SHARED((NPAD, DPAD), _F32),
        pltpu.SemaphoreType.DMA,
    ],
)
def _sc_edge_rows(h_hbm, src_hbm, dst_hbm, ex_hbm, den_hbm, out_hbm,
                  den_v, src_c, dst_c, ex_c, w_v, rows_v, zer_v,
                  acc_sh, sem):
  sidx = lax.axis_index("s")
  wid = sidx

  pltpu.sync_copy(den_hbm.at[pl.ds(0, DEN_R)], den_v)
  pltpu.sync_copy(den_hbm.at[pl.ds(DEN_RP, DEN_R)],
                  rows_v.at[pl.ds(0, DEN_R)])

  def _comb(r, _):
    for q in range(8):
      den_v[r, pl.ds(q * 16, 16)] = (den_v[r, pl.ds(q * 16, 16)]
                                     + rows_v[r, pl.ds(q * 16, 16)])
    return 0
  lax.fori_loop(0, DEN_R, _comb, 0)

  zero16 = jnp.zeros((16,), _F32)
  for r in range(16):
    for q in range(DPAD // 16):
      zer_v[r, pl.ds(q * 16, 16)] = zero16

  stripe = sidx * (NPAD // NS)
  for t in range((NPAD // NS) // 16):
    pltpu.sync_copy(zer_v, acc_sh.at[pl.ds(stripe + t * 16, 16)])
  plsc.subcore_barrier()

  base_row = wid * (ROWS_E // NS)

  def _chunk(ci, _):
    rb = base_row + ci * CB_ROWS
    pltpu.sync_copy(src_hbm.at[pl.ds(rb, CB_ROWS)], src_c)
    pltpu.sync_copy(dst_hbm.at[pl.ds(rb, CB_ROWS)], dst_c)
    pltpu.sync_copy(ex_hbm.at[pl.ds(rb, CB_ROWS)], ex_c)

    descs = []
    for j in range(CB_ROWS):
      descs.append(pltpu.async_copy(
          h_hbm.at[src_c.at[j]], rows_v.at[pl.ds(j * 128, 128)], sem))
    for d in descs:
      d.wait()

    for r in range(CB_ROWS):
      for q in range(8):
        d16 = dst_c[r, pl.ds(q * 16, 16)]
        dv = plsc.load_gather(
            den_v,
            [lax.shift_right_logical(d16, 7), lax.bitwise_and(d16, 127)])
        exv = ex_c[r, pl.ds(q * 16, 16)]
        w_v[pl.ds((r * 8 + q) * 16, 16)] = exv / dv

    def _scale(e, _):
      wb = plsc.load_gather(w_v, [jnp.full((16,), e, _I32)])
      for q in range(DPAD // 16):
        rows_v[e, pl.ds(q * 16, 16)] = rows_v[e, pl.ds(q * 16, 16)] * wb
      return 0
    lax.fori_loop(0, CB_ROWS * 128, _scale, 0)

    for j in range(CB_ROWS):
      pltpu.sync_copy(rows_v.at[pl.ds(j * 128, 128)],
                      acc_sh.at[dst_c.at[j]], add=True)
    return 0

  lax.fori_loop(0, (ROWS_E // NS) // CB_ROWS, _chunk, 0)

  plsc.subcore_barrier()
  pltpu.sync_copy(acc_sh.at[pl.ds(stripe, NPAD // NS)],
                  out_hbm.at[pl.ds(stripe, NPAD // NS)])


def kernel(x, edge_index, W1, a_s1, a_d1, b1, W2, a_s2, a_d2, b2):
  x_pad = jnp.zeros((NPAD, DPAD), _F32).at[:N, :D].set(x)

  loop = jnp.arange(N, dtype=_I32)
  padv = jnp.full((EPAD - NE,), PAD_NODE, _I32)
  src = jnp.concatenate([edge_index[0], loop, padv]).reshape(ROWS_E, 128)
  dst = jnp.concatenate([edge_index[1], loop, padv]).reshape(ROWS_E, 128)

  def pad_w(w):
    return jnp.zeros((DPAD, DPAD), _F32).at[:D, :D].set(w)

  def pad_a8(a_s, a_d):
    return (jnp.zeros((DPAD, 8), _F32)
            .at[:D, 0].set(a_s).at[:D, 1].set(a_d))

  def pad_b(b):
    return jnp.zeros((1, DPAD), _F32).at[0, :D].set(b)

  h1, asd1 = _mat_single(x_pad, pad_w(W1), pad_a8(a_s1, a_d1))
  ex1, den1 = _sc_edge_scalar(asd1, src, dst)
  acc1 = _sc_edge_rows(h1, src, dst, ex1, den1)

  h2, asd2 = _mat_pair(acc1, pad_b(b1), pad_w(W2), pad_a8(a_s2, a_d2))
  ex2, den2 = _sc_edge_scalar(asd2, src, dst)
  acc2 = _sc_edge_rows(h2, src, dst, ex2, den2)

  out = _final_combine(acc2, pad_b(b2))
  return out[:N, :D]


# speedup vs baseline: 25.3077x; 1.5998x over previous
import functools

import jax
import jax.numpy as jnp
from jax import lax
from jax.experimental import pallas as pl
from jax.experimental.pallas import tpu as pltpu
from jax.experimental.pallas import tpu_sc as plsc

N = 10000
D = 101
E = 640000

NPAD = 10240
DPAD = 128
DEN_R = NPAD // 128
DEN_RP = 128

NE = E + N
EPAD = 655360
ROWS_E = EPAD // 128
PAD_NODE = 10016

NC = 2
NS = 16
NW = NC * NS
ET_ROWS = ROWS_E // NW
CA_ROWS = 16
CB_ROWS = 2
R_BLK = 1280

_F32 = jnp.float32
_I32 = jnp.int32


def _mat_body_single(in_ref, w_ref, a8_ref, h_ref, asd_ref):
  s = in_ref[...]
  h = jnp.dot(s, w_ref[...], preferred_element_type=_F32,
              precision=lax.Precision.HIGHEST)
  h_ref[...] = h
  asd_ref[...] = lax.dot_general(
      a8_ref[...], h, (((0,), (1,)), ((), ())),
      preferred_element_type=_F32, precision=lax.Precision.HIGHEST)


def _mat_body_pair(acc_ref, b_ref, w_ref, a8_ref, h_ref, asd_ref):
  s = acc_ref[0] + acc_ref[1] + b_ref[...]
  h = jnp.dot(s, w_ref[...], preferred_element_type=_F32,
              precision=lax.Precision.HIGHEST)
  h_ref[...] = h
  asd_ref[...] = lax.dot_general(
      a8_ref[...], h, (((0,), (1,)), ((), ())),
      preferred_element_type=_F32, precision=lax.Precision.HIGHEST)


def _mat_single(x_pad, w_pad, a8):
  grid = NPAD // R_BLK
  return pl.pallas_call(
      _mat_body_single,
      grid=(grid,),
      in_specs=[
          pl.BlockSpec((R_BLK, DPAD), lambda i: (i, 0)),
          pl.BlockSpec((DPAD, DPAD), lambda i: (0, 0)),
          pl.BlockSpec((DPAD, 8), lambda i: (0, 0)),
      ],
      out_specs=[
          pl.BlockSpec((R_BLK, DPAD), lambda i: (i, 0)),
          pl.BlockSpec((8, R_BLK), lambda i: (0, i)),
      ],
      out_shape=[
          jax.ShapeDtypeStruct((NPAD, DPAD), _F32),
          jax.ShapeDtypeStruct((8, NPAD), _F32),
      ],
  )(x_pad, w_pad, a8)


def _mat_pair(acc, b_pad, w_pad, a8):
  grid = NPAD // R_BLK
  return pl.pallas_call(
      _mat_body_pair,
      grid=(grid,),
      in_specs=[
          pl.BlockSpec((2, R_BLK, DPAD), lambda i: (0, i, 0)),
          pl.BlockSpec((1, DPAD), lambda i: (0, 0)),
          pl.BlockSpec((DPAD, DPAD), lambda i: (0, 0)),
          pl.BlockSpec((DPAD, 8), lambda i: (0, 0)),
      ],
      out_specs=[
          pl.BlockSpec((R_BLK, DPAD), lambda i: (i, 0)),
          pl.BlockSpec((8, R_BLK), lambda i: (0, i)),
      ],
      out_shape=[
          jax.ShapeDtypeStruct((NPAD, DPAD), _F32),
          jax.ShapeDtypeStruct((8, NPAD), _F32),
      ],
  )(acc, b_pad, w_pad, a8)


def _final_body(acc_ref, b_ref, out_ref):
  out_ref[...] = acc_ref[0] + acc_ref[1] + b_ref[...]


def _final_combine(acc, b_pad):
  grid = NPAD // R_BLK
  return pl.pallas_call(
      _final_body,
      grid=(grid,),
      in_specs=[
          pl.BlockSpec((2, R_BLK, DPAD), lambda i: (0, i, 0)),
          pl.BlockSpec((1, DPAD), lambda i: (0, 0)),
      ],
      out_specs=pl.BlockSpec((R_BLK, DPAD), lambda i: (i, 0)),
      out_shape=jax.ShapeDtypeStruct((NPAD, DPAD), _F32),
  )(acc, b_pad)


_SC_MESH = plsc.VectorSubcoreMesh(core_axis_name="c", subcore_axis_name="s")


@functools.partial(
    pl.kernel,
    out_type=(
        jax.ShapeDtypeStruct((ROWS_E, 128), _F32),
        jax.ShapeDtypeStruct((NC * DEN_RP, 128), _F32),
    ),
    mesh=_SC_MESH,
    compiler_params=pltpu.CompilerParams(needs_layout_passes=False),
    scratch_types=[
        pltpu.VMEM((NPAD,), _F32),
        pltpu.VMEM((NPAD,), _F32),
        pltpu.VMEM((DEN_RP, 128), _F32),
        pltpu.VMEM((1, 128), _I32),
        pltpu.VMEM((CA_ROWS, 128), _I32),
        pltpu.VMEM((CA_ROWS, 128), _I32),
        pltpu.VMEM((CA_ROWS, 128), _F32),
        pltpu.VMEM_SHARED((DEN_RP, 128), _F32),
    ],
)
def _sc_edge_scalar(asd_hbm, src_hbm, dst_hbm, ex_hbm, den_hbm,
                    as_v, ad_v, den_v, ramp_v, src_c, dst_c, ex_c, den_sh):
  cidx = lax.axis_index("c")
  sidx = lax.axis_index("s")
  wid = sidx * NC + cidx

  pltpu.sync_copy(asd_hbm.at[0], as_v)
  pltpu.sync_copy(asd_hbm.at[1], ad_v)

  zero16 = jnp.zeros((16,), _F32)

  def _zero_row(r, _):
    for q in range(8):
      den_v[r, pl.ds(q * 16, 16)] = zero16
    return 0
  lax.fori_loop(0, DEN_RP, _zero_row, 0)

  iota16 = lax.iota(_I32, 16)
  for q in range(8):
    ramp_v[0, pl.ds(q * 16, 16)] = iota16 + q * 16

  base_row = wid * ET_ROWS
  for ci in range(ET_ROWS // CA_ROWS):
    rb = base_row + ci * CA_ROWS
    pltpu.sync_copy(src_hbm.at[pl.ds(rb, CA_ROWS)], src_c)
    pltpu.sync_copy(dst_hbm.at[pl.ds(rb, CA_ROWS)], dst_c)

    def _row(r, _):
      for q in range(8):
        s16 = src_c[r, pl.ds(q * 16, 16)]
        d16 = dst_c[r, pl.ds(q * 16, 16)]
        av = plsc.load_gather(as_v, [s16])
        bv = plsc.load_gather(ad_v, [d16])
        e = av + bv
        e = jnp.where(e > 0.0, e, 0.2 * e)
        exv = jnp.exp(e)
        ex_c[r, pl.ds(q * 16, 16)] = exv
        plsc.addupdate_scatter(
            den_v,
            [lax.shift_right_logical(d16, 7), lax.bitwise_and(d16, 127)],
            exv)
      return 0
    lax.fori_loop(0, CA_ROWS, _row, 0)

    pltpu.sync_copy(ex_c, ex_hbm.at[pl.ds(rb, CA_ROWS)])

  @pl.when(sidx == 0)
  def _():
    pltpu.sync_copy(den_v, den_sh)
  plsc.subcore_barrier()

  @pl.when(sidx != 0)
  def _():
    pltpu.sync_copy(den_v, den_sh.at[ramp_v.at[0]], add=True)
  plsc.subcore_barrier()

  @pl.when(sidx == 0)
  def _():
    pltpu.sync_copy(den_sh, den_hbm.at[pl.ds(cidx * DEN_RP, DEN_RP)])


@functools.partial(
    pl.kernel,
    out_type=jax.ShapeDtypeStruct((NC, NPAD, DPAD), _F32),
    mesh=_SC_MESH,
    compiler_params=pltpu.CompilerParams(needs_layout_passes=False),
    scratch_types=[
        pltpu.VMEM((DEN_R, 128), _F32),
        pltpu.VMEM((CB_ROWS, 128), _I32),
        pltpu.VMEM((CB_ROWS, 128), _I32),
        pltpu.VMEM((CB_ROWS, 128), _F32),
        pltpu.VMEM((CB_ROWS * 128,), _F32),
        pltpu.VMEM((CB_ROWS * 128, DPAD), _F32),
        pltpu.VMEM((16, DPAD), _F32),
        pltpu.VMEM_SHARED((NPAD, DPAD), _F32),
        pltpu.SemaphoreType.DMA,
    ],
)
def _sc_edge_rows(h_hbm, src_hbm, dst_hbm, ex_hbm, den_hbm, out_hbm,
                  den_v, src_c, dst_c, ex_c, w_v, rows_v, zer_v,
                  acc_sh, sem):
  cidx = lax.axis_index("c")
  sidx = lax.axis_index("s")
  wid = sidx * NC + cidx

  pltpu.sync_copy(den_hbm.at[pl.ds(0, DEN_R)], den_v)
  pltpu.sync_copy(den_hbm.at[pl.ds(DEN_RP, DEN_R)],
                  rows_v.at[pl.ds(0, DEN_R)])

  def _comb(r, _):
    for q in range(8):
      den_v[r, pl.ds(q * 16, 16)] = (den_v[r, pl.ds(q * 16, 16)]
                                     + rows_v[r, pl.ds(q * 16, 16)])
    return 0
  lax.fori_loop(0, DEN_R, _comb, 0)

  zero16 = jnp.zeros((16,), _F32)
  for r in range(16):
    for q in range(DPAD // 16):
      zer_v[r, pl.ds(q * 16, 16)] = zero16

  stripe = sidx * (NPAD // NS)
  for t in range((NPAD // NS) // 16):
    pltpu.sync_copy(zer_v, acc_sh.at[pl.ds(stripe + t * 16, 16)])
  plsc.subcore_barrier()

  base_row = wid * ET_ROWS

  def _chunk(ci, _):
    rb = base_row + ci * CB_ROWS
    pltpu.sync_copy(src_hbm.at[pl.ds(rb, CB_ROWS)], src_c)
    pltpu.sync_copy(dst_hbm.at[pl.ds(rb, CB_ROWS)], dst_c)
    pltpu.sync_copy(ex_hbm.at[pl.ds(rb, CB_ROWS)], ex_c)

    descs = []
    for j in range(CB_ROWS):
      descs.append(pltpu.async_copy(
          h_hbm.at[src_c.at[j]], rows_v.at[pl.ds(j * 128, 128)], sem))
    for d in descs:
      d.wait()

    for r in range(CB_ROWS):
      for q in range(8):
        d16 = dst_c[r, pl.ds(q * 16, 16)]
        dv = plsc.load_gather(
            den_v,
            [lax.shift_right_logical(d16, 7), lax.bitwise_and(d16, 127)])
        exv = ex_c[r, pl.ds(q * 16, 16)]
        w_v[pl.ds((r * 8 + q) * 16, 16)] = exv / dv

    def _scale(e, _):
      wb = plsc.load_gather(w_v, [jnp.full((16,), e, _I32)])
      for q in range(DPAD // 16):
        rows_v[e, pl.ds(q * 16, 16)] = rows_v[e, pl.ds(q * 16, 16)] * wb
      return 0
    lax.fori_loop(0, CB_ROWS * 128, _scale, 0)

    for j in range(CB_ROWS):
      pltpu.sync_copy(rows_v.at[pl.ds(j * 128, 128)],
                      acc_sh.at[dst_c.at[j]], add=True)
    return 0

  lax.fori_loop(0, ET_ROWS // CB_ROWS, _chunk, 0)

  plsc.subcore_barrier()
  pltpu.sync_copy(acc_sh.at[pl.ds(stripe, NPAD // NS)],
                  out_hbm.at[cidx, pl.ds(stripe, NPAD // NS)])


def kernel(x, edge_index, W1, a_s1, a_d1, b1, W2, a_s2, a_d2, b2):
  x_pad = jnp.zeros((NPAD, DPAD), _F32).at[:N, :D].set(x)

  loop = jnp.arange(N, dtype=_I32)
  padv = jnp.full((EPAD - NE,), PAD_NODE, _I32)
  src = jnp.concatenate([edge_index[0], loop, padv]).reshape(ROWS_E, 128)
  dst = jnp.concatenate([edge_index[1], loop, padv]).reshape(ROWS_E, 128)

  def pad_w(w):
    return jnp.zeros((DPAD, DPAD), _F32).at[:D, :D].set(w)

  def pad_a8(a_s, a_d):
    return (jnp.zeros((DPAD, 8), _F32)
            .at[:D, 0].set(a_s).at[:D, 1].set(a_d))

  def pad_b(b):
    return jnp.zeros((1, DPAD), _F32).at[0, :D].set(b)

  h1, asd1 = _mat_single(x_pad, pad_w(W1), pad_a8(a_s1, a_d1))
  ex1, den1 = _sc_edge_scalar(asd1, src, dst)
  acc1 = _sc_edge_rows(h1, src, dst, ex1, den1)

  h2, asd2 = _mat_pair(acc1, pad_b(b1), pad_w(W2), pad_a8(a_s2, a_d2))
  ex2, den2 = _sc_edge_scalar(asd2, src, dst)
  acc2 = _sc_edge_rows(h2, src, dst, ex2, den2)

  out = _final_combine(acc2, pad_b(b2))
  return out[:N, :D]


# speedup vs baseline: 31.9809x; 1.2637x over previous
import functools

import jax
import jax.numpy as jnp
from jax import lax
from jax.experimental import pallas as pl
from jax.experimental.pallas import tpu as pltpu
from jax.experimental.pallas import tpu_sc as plsc

N = 10000
D = 101
E = 640000

NPAD = 10240
DPAD = 128
DEN_R = NPAD // 128
DEN_RP = 128

NE = E + N
EPAD = 655360
ROWS_E = EPAD // 128
PAD_NODE = 10016

NC = 2
NS = 16
NW = NC * NS
ET_ROWS = ROWS_E // NW
CA_ROWS = 16
CB_ROWS = 2
R_BLK = 1280

_F32 = jnp.float32
_I32 = jnp.int32


def _mat_body_single(in_ref, w_ref, a8_ref, h_ref, asd_ref):
  s = in_ref[...]
  h = jnp.dot(s, w_ref[...], preferred_element_type=_F32,
              precision=lax.Precision.HIGHEST)
  h_ref[...] = h
  asd_ref[...] = lax.dot_general(
      a8_ref[...], h, (((0,), (1,)), ((), ())),
      preferred_element_type=_F32, precision=lax.Precision.HIGHEST)


def _mat_body_pair(acc_ref, b_ref, w_ref, a8_ref, h_ref, asd_ref):
  s = acc_ref[0] + acc_ref[1] + b_ref[...]
  h = jnp.dot(s, w_ref[...], preferred_element_type=_F32,
              precision=lax.Precision.HIGHEST)
  h_ref[...] = h
  asd_ref[...] = lax.dot_general(
      a8_ref[...], h, (((0,), (1,)), ((), ())),
      preferred_element_type=_F32, precision=lax.Precision.HIGHEST)


def _mat_single(x_pad, w_pad, a8):
  grid = NPAD // R_BLK
  return pl.pallas_call(
      _mat_body_single,
      grid=(grid,),
      in_specs=[
          pl.BlockSpec((R_BLK, DPAD), lambda i: (i, 0)),
          pl.BlockSpec((DPAD, DPAD), lambda i: (0, 0)),
          pl.BlockSpec((DPAD, 8), lambda i: (0, 0)),
      ],
      out_specs=[
          pl.BlockSpec((R_BLK, DPAD), lambda i: (i, 0)),
          pl.BlockSpec((8, R_BLK), lambda i: (0, i)),
      ],
      out_shape=[
          jax.ShapeDtypeStruct((NPAD, DPAD), _F32),
          jax.ShapeDtypeStruct((8, NPAD), _F32),
      ],
  )(x_pad, w_pad, a8)


def _mat_pair(acc, b_pad, w_pad, a8):
  grid = NPAD // R_BLK
  return pl.pallas_call(
      _mat_body_pair,
      grid=(grid,),
      in_specs=[
          pl.BlockSpec((2, R_BLK, DPAD), lambda i: (0, i, 0)),
          pl.BlockSpec((1, DPAD), lambda i: (0, 0)),
          pl.BlockSpec((DPAD, DPAD), lambda i: (0, 0)),
          pl.BlockSpec((DPAD, 8), lambda i: (0, 0)),
      ],
      out_specs=[
          pl.BlockSpec((R_BLK, DPAD), lambda i: (i, 0)),
          pl.BlockSpec((8, R_BLK), lambda i: (0, i)),
      ],
      out_shape=[
          jax.ShapeDtypeStruct((NPAD, DPAD), _F32),
          jax.ShapeDtypeStruct((8, NPAD), _F32),
      ],
  )(acc, b_pad, w_pad, a8)


def _final_body(acc_ref, b_ref, out_ref):
  out_ref[...] = acc_ref[0] + acc_ref[1] + b_ref[...]


def _final_combine(acc, b_pad):
  grid = NPAD // R_BLK
  return pl.pallas_call(
      _final_body,
      grid=(grid,),
      in_specs=[
          pl.BlockSpec((2, R_BLK, DPAD), lambda i: (0, i, 0)),
          pl.BlockSpec((1, DPAD), lambda i: (0, 0)),
      ],
      out_specs=pl.BlockSpec((R_BLK, DPAD), lambda i: (i, 0)),
      out_shape=jax.ShapeDtypeStruct((NPAD, DPAD), _F32),
  )(acc, b_pad)


_SC_MESH = plsc.VectorSubcoreMesh(core_axis_name="c", subcore_axis_name="s")


@functools.partial(
    pl.kernel,
    out_type=(
        jax.ShapeDtypeStruct((ROWS_E, 128), _F32),
        jax.ShapeDtypeStruct((NC * DEN_RP, 128), _F32),
    ),
    mesh=_SC_MESH,
    compiler_params=pltpu.CompilerParams(needs_layout_passes=False),
    scratch_types=[
        pltpu.VMEM((NPAD,), _F32),
        pltpu.VMEM((NPAD,), _F32),
        pltpu.VMEM((DEN_RP, 128), _F32),
        pltpu.VMEM((1, 128), _I32),
        pltpu.VMEM((CA_ROWS, 2, 128), _I32),
        pltpu.VMEM((CA_ROWS, 128), _F32),
        pltpu.VMEM_SHARED((DEN_RP, 128), _F32),
    ],
)
def _sc_edge_scalar(asd_hbm, sd_hbm, ex_hbm, den_hbm,
                    as_v, ad_v, den_v, ramp_v, sd_c, ex_c, den_sh):
  cidx = lax.axis_index("c")
  sidx = lax.axis_index("s")
  wid = sidx * NC + cidx

  pltpu.sync_copy(asd_hbm.at[0], as_v)
  pltpu.sync_copy(asd_hbm.at[1], ad_v)

  zero16 = jnp.zeros((16,), _F32)

  def _zero_row(r, _):
    for q in range(8):
      den_v[r, pl.ds(q * 16, 16)] = zero16
    return 0
  lax.fori_loop(0, DEN_RP, _zero_row, 0)

  iota16 = lax.iota(_I32, 16)
  for q in range(8):
    ramp_v[0, pl.ds(q * 16, 16)] = iota16 + q * 16

  base_row = wid * ET_ROWS
  for ci in range(ET_ROWS // CA_ROWS):
    rb = base_row + ci * CA_ROWS
    pltpu.sync_copy(sd_hbm.at[pl.ds(rb, CA_ROWS)], sd_c)

    def _row(r, _):
      for q in range(8):
        s16 = sd_c[r, 0, pl.ds(q * 16, 16)]
        d16 = sd_c[r, 1, pl.ds(q * 16, 16)]
        av = plsc.load_gather(as_v, [s16])
        bv = plsc.load_gather(ad_v, [d16])
        e = av + bv
        e = jnp.where(e > 0.0, e, 0.2 * e)
        exv = jnp.exp(e)
        ex_c[r, pl.ds(q * 16, 16)] = exv
        plsc.addupdate_scatter(
            den_v,
            [lax.shift_right_logical(d16, 7), lax.bitwise_and(d16, 127)],
            exv)
      return 0
    lax.fori_loop(0, CA_ROWS, _row, 0)

    pltpu.sync_copy(ex_c, ex_hbm.at[pl.ds(rb, CA_ROWS)])

  @pl.when(sidx == 0)
  def _():
    pltpu.sync_copy(den_v, den_sh)
  plsc.subcore_barrier()

  @pl.when(sidx != 0)
  def _():
    pltpu.sync_copy(den_v, den_sh.at[ramp_v.at[0]], add=True)
  plsc.subcore_barrier()

  @pl.when(sidx == 0)
  def _():
    pltpu.sync_copy(den_sh, den_hbm.at[pl.ds(cidx * DEN_RP, DEN_RP)])


@functools.partial(
    pl.kernel,
    out_type=jax.ShapeDtypeStruct((NC, NPAD, DPAD), _F32),
    mesh=_SC_MESH,
    compiler_params=pltpu.CompilerParams(needs_layout_passes=False),
    scratch_types=[
        pltpu.VMEM((DEN_R, 128), _F32),
        pltpu.VMEM((2, 2, 128), _I32),
        pltpu.VMEM((2, 128), _F32),
        pltpu.VMEM((2, 128), _F32),
        pltpu.VMEM((2 * 128, DPAD), _F32),
        pltpu.VMEM((16, DPAD), _F32),
        pltpu.VMEM_SHARED((NPAD, DPAD), _F32),
        pltpu.SemaphoreType.DMA,
        pltpu.SemaphoreType.DMA,
        pltpu.SemaphoreType.DMA,
        pltpu.SemaphoreType.DMA,
    ],
)
def _sc_edge_rows(h_hbm, sd_hbm, ex_hbm, den_hbm, out_hbm,
                  den_v, sd_c, ex_c, w_v, rows_v, zer_v,
                  acc_sh, semg0, semg1, sems0, sems1):
  cidx = lax.axis_index("c")
  sidx = lax.axis_index("s")
  wid = sidx * NC + cidx

  pltpu.sync_copy(den_hbm.at[pl.ds(0, DEN_R)], den_v)
  pltpu.sync_copy(den_hbm.at[pl.ds(DEN_RP, DEN_R)],
                  rows_v.at[pl.ds(0, DEN_R)])

  def _comb(r, _):
    for q in range(8):
      den_v[r, pl.ds(q * 16, 16)] = (den_v[r, pl.ds(q * 16, 16)]
                                     + rows_v[r, pl.ds(q * 16, 16)])
    return 0
  lax.fori_loop(0, DEN_R, _comb, 0)

  zero16 = jnp.zeros((16,), _F32)
  for r in range(16):
    for q in range(DPAD // 16):
      zer_v[r, pl.ds(q * 16, 16)] = zero16

  stripe = sidx * (NPAD // NS)
  for t in range((NPAD // NS) // 16):
    pltpu.sync_copy(zer_v, acc_sh.at[pl.ds(stripe + t * 16, 16)])
  plsc.subcore_barrier()

  base_row = wid * ET_ROWS

  def _loadidx(p, row):
    pltpu.sync_copy(sd_hbm.at[row], sd_c.at[p])
    pltpu.sync_copy(ex_hbm.at[row], ex_c.at[p])

  def _startg(p, semg):
    return pltpu.async_copy(h_hbm.at[sd_c.at[p, 0]],
                            rows_v.at[pl.ds(p * 128, 128)], semg)

  def _starts(p, sems):
    return pltpu.async_copy(rows_v.at[pl.ds(p * 128, 128)],
                            acc_sh.at[sd_c.at[p, 1]], sems, add=True)

  def _drain(sem, p):
    pltpu.make_async_copy(h_hbm.at[pl.ds(0, 128)],
                          rows_v.at[pl.ds(p * 128, 128)], sem).wait()

  def _process(p):
    pfull = jnp.full((16,), p, _I32)
    for q in range(8):
      d16 = sd_c[p, 1, pl.ds(q * 16, 16)]
      dv = plsc.load_gather(
          den_v,
          [lax.shift_right_logical(d16, 7), lax.bitwise_and(d16, 127)])
      exv = ex_c[p, pl.ds(q * 16, 16)]
      w_v[p, pl.ds(q * 16, 16)] = exv / dv

    def _scale(e4, _):
      for u in range(4):
        e = e4 * 4 + u
        wb = plsc.load_gather(w_v, [pfull, jnp.full((16,), e, _I32)])
        r = p * 128 + e
        for q in range(DPAD // 16):
          rows_v[r, pl.ds(q * 16, 16)] = rows_v[r, pl.ds(q * 16, 16)] * wb
      return 0
    lax.fori_loop(0, 32, _scale, 0)

  _loadidx(0, base_row)
  _startg(0, semg0)
  pltpu.async_copy(h_hbm.at[pl.ds(0, 128)],
                   rows_v.at[pl.ds(128, 128)], sems1)

  def _body(ci, _):
    a = base_row + 2 * ci
    _drain(sems1, 1)
    _loadidx(1, a + 1)
    gb = _startg(1, semg1)
    _drain(semg0, 0)
    _process(0)
    sa = _starts(0, sems0)
    gb.wait()
    _process(1)
    _starts(1, sems1)
    sa.wait()
    nxt = jnp.where(ci < ET_ROWS // 2 - 1, a + 2, base_row)
    _loadidx(0, nxt)
    _startg(0, semg0)
    return 0

  lax.fori_loop(0, ET_ROWS // 2, _body, 0)
  _drain(semg0, 0)
  _drain(sems1, 1)

  plsc.subcore_barrier()
  pltpu.sync_copy(acc_sh.at[pl.ds(stripe, NPAD // NS)],
                  out_hbm.at[cidx, pl.ds(stripe, NPAD // NS)])


def kernel(x, edge_index, W1, a_s1, a_d1, b1, W2, a_s2, a_d2, b2):
  x_pad = jnp.zeros((NPAD, DPAD), _F32).at[:N, :D].set(x)

  loop = jnp.arange(N, dtype=_I32)
  padv = jnp.full((EPAD - NE,), PAD_NODE, _I32)
  src = jnp.concatenate([edge_index[0], loop, padv]).reshape(ROWS_E, 128)
  dst = jnp.concatenate([edge_index[1], loop, padv]).reshape(ROWS_E, 128)
  sd = jnp.stack([src, dst], axis=1)

  def pad_w(w):
    return jnp.zeros((DPAD, DPAD), _F32).at[:D, :D].set(w)

  def pad_a8(a_s, a_d):
    return (jnp.zeros((DPAD, 8), _F32)
            .at[:D, 0].set(a_s).at[:D, 1].set(a_d))

  def pad_b(b):
    return jnp.zeros((1, DPAD), _F32).at[0, :D].set(b)

  h1, asd1 = _mat_single(x_pad, pad_w(W1), pad_a8(a_s1, a_d1))
  ex1, den1 = _sc_edge_scalar(asd1, sd)
  acc1 = _sc_edge_rows(h1, sd, ex1, den1)

  h2, asd2 = _mat_pair(acc1, pad_b(b1), pad_w(W2), pad_a8(a_s2, a_d2))
  ex2, den2 = _sc_edge_scalar(asd2, sd)
  acc2 = _sc_edge_rows(h2, sd, ex2, den2)

  out = _final_combine(acc2, pad_b(b2))
  return out[:N, :D]
